# baseline (speedup 1.0000x reference)
import jax
import jax.numpy as jnp
from jax import lax
from jax.experimental import pallas as pl
from jax.experimental.pallas import tpu as pltpu
from jax.experimental.pallas import tpu_sc as plsc

_B = 4096
_KB = 1000
_KB_PAD = 1024
_TOPK = 5
_TEMP_INV = 1.0 / 0.07
_BLK = 1024
_NBLK = _B // _BLK
_NEG = -1e30

_NW = 32
_RPW = _B // _NW
_CHUNK = 16
_NCH = _RPW // _CHUNK


def _prep_kern(keys_ref, sem_ref, val_ref, kn_ref, vs_ref):
    k = keys_ref[...]
    kn = k * (1.0 / jnp.maximum(
        jnp.sqrt(jnp.sum(k * k, axis=-1, keepdims=True)), 1e-12))
    kn_ref[...] = jnp.zeros_like(kn_ref)
    kn_ref[:_KB, :] = kn
    sm = sem_ref[...]
    semn = sm * (1.0 / jnp.maximum(
        jnp.sqrt(jnp.sum(sm * sm, axis=-1, keepdims=True)), 1e-12))
    vs_ref[...] = jnp.zeros_like(vs_ref)
    vs_ref[:_KB, :512] = val_ref[...]
    vs_ref[:_KB, 512:] = semn


def _tc_kern(u_ref, c_ref, w_ref, b_ref, kn_ref, idx_ref, al_ref):
    w = w_ref[...]
    nt = (((1,), (1,)), ((), ()))
    q = (lax.dot_general(u_ref[...], w[:, :512], nt,
                         preferred_element_type=jnp.float32)
         + lax.dot_general(c_ref[...], w[:, 512:], nt,
                           preferred_element_type=jnp.float32)
         + b_ref[...])
    q = jnp.maximum(q, 0.0)
    qn = q * (1.0 / jnp.maximum(
        jnp.sqrt(jnp.sum(q * q, axis=-1, keepdims=True)), 1e-12))

    s = lax.dot_general(qn, kn_ref[...], nt,
                        preferred_element_type=jnp.float32) * _TEMP_INV
    colf = lax.broadcasted_iota(jnp.int32, s.shape, 1).astype(jnp.float32)
    s = jnp.where(colf < _KB, s, _NEG)

    idx_cols = []
    m_cols = []
    s_cur = s
    for j in range(_TOPK):
        m = jnp.max(s_cur, axis=-1, keepdims=True)
        idxf = jnp.min(jnp.where(s_cur == m, colf, 3e9),
                       axis=-1, keepdims=True)
        idx_cols.append(idxf)
        m_cols.append(m)
        s_cur = jnp.where(colf == idxf, _NEG, s_cur)

    m0 = m_cols[0]
    ws = [jnp.exp(m - m0) for m in m_cols]
    rdenom = 1.0 / (ws[0] + ws[1] + ws[2] + ws[3] + ws[4])
    ones16 = jnp.ones((s.shape[0], 16), jnp.float32)
    al_ref[...] = jnp.concatenate(
        [(wj * rdenom) * ones16 for wj in ws]
        + [jnp.zeros((s.shape[0], 48), jnp.float32)], axis=1)
    idx_ref[...] = jnp.concatenate(
        [f.astype(jnp.int32) for f in idx_cols]
        + [jnp.zeros((s.shape[0], 8 - _TOPK), jnp.int32)], axis=1)


def _sc_kern(vs_hbm, idx_hbm, al_hbm, e_hbm, g_hbm,
             idx_v, al_v, rows_v, e_v, g_v, sem):
    wid = lax.axis_index("s") * 2 + lax.axis_index("c")
    base = wid * _RPW
    pltpu.sync_copy(idx_hbm.at[wid], idx_v)
    pltpu.sync_copy(al_hbm.at[pl.ds(base, _RPW)], al_v)

    def chunk_body(ch, carry):
        pltpu.async_copy(
            vs_hbm.at[idx_v.at[pl.ds(ch * (_CHUNK * _TOPK),
                                     _CHUNK * _TOPK)]],
            rows_v, sem).wait()

        def row_body(jj, carry2):
            r5 = jj * _TOPK
            arow = ch * _CHUNK + jj
            aks = [al_v[arow, pl.ds(k * 16, 16)] for k in range(_TOPK)]
            for v in range(32):
                de = pl.ds(v * 16, 16)
                dg = pl.ds(512 + v * 16, 16)
                acc_e = aks[0] * rows_v[r5, de]
                acc_g = aks[0] * rows_v[r5, dg]
                for k in range(1, _TOPK):
                    acc_e = acc_e + aks[k] * rows_v[r5 + k, de]
                    acc_g = acc_g + aks[k] * rows_v[r5 + k, dg]
                e_v[jj, de] = acc_e
                g_v[jj, pl.ds(v * 16, 16)] = acc_g
            return carry2

        lax.fori_loop(0, _CHUNK, row_body, 0)
        rb = base + ch * _CHUNK
        pltpu.sync_copy(e_v, e_hbm.at[pl.ds(rb, _CHUNK)])
        pltpu.sync_copy(g_v, g_hbm.at[pl.ds(rb, _CHUNK)])
        return carry

    lax.fori_loop(0, _NCH, chunk_body, 0)


def _cost_kern(c_ref, g_ref, cost_ref):
    i = pl.program_id(0)
    c = c_ref[...]
    csn = c * (1.0 / jnp.maximum(
        jnp.sqrt(jnp.sum(c * c, axis=-1, keepdims=True)), 1e-12))
    part = jnp.sum(1.0 - jnp.sum(csn * g_ref[...], axis=-1)) * (1.0 / _B)

    @pl.when(i == 0)
    def _():
        cost_ref[...] = jnp.zeros_like(cost_ref)

    cost_ref[...] += part


@jax.jit
def kernel(u_X, c_S, W, b, keys, values, semantic_embeddings):
    b2 = b.reshape(1, -1)
    dk = keys.shape[1]
    dv = values.shape[1]
    dsem = semantic_embeddings.shape[1]

    kn, vs = pl.pallas_call(
        _prep_kern,
        out_shape=[
            jax.ShapeDtypeStruct((_KB_PAD, dk), jnp.float32),
            jax.ShapeDtypeStruct((_KB_PAD, dv + dsem), jnp.float32),
        ],
    )(keys, semantic_embeddings, values)

    idx_out, al_out = pl.pallas_call(
        _tc_kern,
        grid=(_NBLK,),
        in_specs=[
            pl.BlockSpec((_BLK, u_X.shape[1]), lambda i: (i, 0)),
            pl.BlockSpec((_BLK, c_S.shape[1]), lambda i: (i, 0)),
            pl.BlockSpec(W.shape, lambda i: (0, 0)),
            pl.BlockSpec((1, W.shape[0]), lambda i: (0, 0)),
            pl.BlockSpec((_KB_PAD, dk), lambda i: (0, 0)),
        ],
        out_specs=[
            pl.BlockSpec((_BLK, 8), lambda i: (i, 0)),
            pl.BlockSpec((_BLK, 128), lambda i: (i, 0)),
        ],
        out_shape=[
            jax.ShapeDtypeStruct((_B, 8), jnp.int32),
            jax.ShapeDtypeStruct((_B, 128), jnp.float32),
        ],
    )(u_X, c_S, W, b2, kn)

    idx_flat = idx_out[:, :_TOPK].reshape(_NW, _RPW * _TOPK)

    mesh = plsc.VectorSubcoreMesh(core_axis_name="c", subcore_axis_name="s")
    e_out, g_out = pl.kernel(
        _sc_kern,
        mesh=mesh,
        out_type=[
            jax.ShapeDtypeStruct((_B, dv), jnp.float32),
            jax.ShapeDtypeStruct((_B, dsem), jnp.float32),
        ],
        scratch_types=[
            pltpu.VMEM((_RPW * _TOPK,), jnp.int32),
            pltpu.VMEM((_RPW, 128), jnp.float32),
            pltpu.VMEM((_CHUNK * _TOPK, dv + dsem), jnp.float32),
            pltpu.VMEM((_CHUNK, dv), jnp.float32),
            pltpu.VMEM((_CHUNK, dsem), jnp.float32),
            pltpu.SemaphoreType.DMA,
        ],
    )(vs, idx_flat, al_out)

    cost_out = pl.pallas_call(
        _cost_kern,
        grid=(4,),
        in_specs=[
            pl.BlockSpec((_B // 4, c_S.shape[1]), lambda i: (i, 0)),
            pl.BlockSpec((_B // 4, dsem), lambda i: (i, 0)),
        ],
        out_specs=pl.BlockSpec((1, 1), lambda i: (0, 0)),
        out_shape=jax.ShapeDtypeStruct((1, 1), jnp.float32),
    )(c_S, g_out)

    return (e_out, idx_out[:, :_TOPK], cost_out[0, 0])


# speedup vs baseline: 3.5894x; 3.5894x over previous
import jax
import jax.numpy as jnp
from jax.experimental import pallas as pl
from jax.experimental.pallas import tpu as pltpu

_B = 4096
_KB = 1000
_KB_PAD = 1024
_TOPK = 5
_TEMP_INV = 1.0 / 0.07
_BLK = 1024
_NBLK = _B // _BLK
_NEG = -1e30


def _main_kern(u_ref, c_ref, w_ref, b_ref, keys_ref, sem_ref, val_ref,
               e_ref, idx_ref, cost_ref, kn_ref, vs_ref, s_ref, csn_ref):
    i = pl.program_id(0)

    @pl.when(i == 0)
    def _():
        k = keys_ref[...]
        kn = k * (1.0 / jnp.maximum(
            jnp.sqrt(jnp.sum(k * k, axis=-1, keepdims=True)), 1e-12))
        kn_ref[...] = jnp.zeros_like(kn_ref)
        kn_ref[:_KB, :] = kn
        sm = sem_ref[...]
        semn = sm * (1.0 / jnp.maximum(
            jnp.sqrt(jnp.sum(sm * sm, axis=-1, keepdims=True)), 1e-12))
        vs_ref[...] = jnp.zeros_like(vs_ref)
        vs_ref[:_KB, :512] = val_ref[...]
        vs_ref[:_KB, 512:] = semn
        cost_ref[...] = jnp.zeros_like(cost_ref)

    nt = (((1,), (1,)), ((), ()))

    @pl.when(i < _NBLK)
    def _():
        w = w_ref[...]
        c = c_ref[...]
        q = (jax.lax.dot_general(u_ref[...], w[:, :512], nt,
                                 preferred_element_type=jnp.float32)
             + jax.lax.dot_general(c, w[:, 512:], nt,
                                   preferred_element_type=jnp.float32)
             + b_ref[...])
        q = jnp.maximum(q, 0.0)
        qn = q * (1.0 / jnp.maximum(
            jnp.sqrt(jnp.sum(q * q, axis=-1, keepdims=True)), 1e-12))
        s_ref[i % 2] = jax.lax.dot_general(
            qn, kn_ref[...], nt,
            preferred_element_type=jnp.float32) * _TEMP_INV
        csn_ref[i % 2] = c * (1.0 / jnp.maximum(
            jnp.sqrt(jnp.sum(c * c, axis=-1, keepdims=True)), 1e-12))

    @pl.when(i > 0)
    def _():
        s = s_ref[(i - 1) % 2]
        colf = jax.lax.broadcasted_iota(
            jnp.int32, s.shape, 1).astype(jnp.float32)
        s = jnp.where(colf < _KB, s, _NEG)

        idx_cols = []
        m0 = None
        denom = None
        s_cur = s
        for j in range(_TOPK):
            m = jnp.max(s_cur, axis=-1, keepdims=True)
            idxf = jnp.min(jnp.where(s_cur == m, colf, 3e9),
                           axis=-1, keepdims=True)
            if j == 0:
                m0 = m
                denom = jnp.ones_like(m)
            else:
                denom = denom + jnp.exp(m - m0)
            idx_cols.append(idxf)
            s_cur = jnp.where(colf == idxf, _NEG, s_cur)

        selected = (s_cur == _NEG) & (colf < _KB)
        walpha = jnp.where(selected, jnp.exp(s - m0) * (1.0 / denom), 0.0)

        eg = jax.lax.dot_general(
            walpha, vs_ref[...], (((1,), (0,)), ((), ())),
            preferred_element_type=jnp.float32)
        dv = e_ref.shape[1]
        e_ref[...] = eg[:, :dv]
        g = eg[:, dv:]
        idx_ref[...] = jnp.concatenate(
            [f.astype(jnp.int32) for f in idx_cols]
            + [jnp.zeros((s.shape[0], 8 - _TOPK), jnp.int32)], axis=1)

        csn = csn_ref[(i - 1) % 2]
        part = jnp.sum(1.0 - jnp.sum(csn * g, axis=-1)) * (1.0 / _B)
        cost_ref[...] += part


@jax.jit
def kernel(u_X, c_S, W, b, keys, values, semantic_embeddings):
    b2 = b.reshape(1, -1)
    dk = keys.shape[1]
    dv = values.shape[1]
    dsem = semantic_embeddings.shape[1]
    last = _NBLK - 1

    e_out, idx_out, cost_out = pl.pallas_call(
        _main_kern,
        grid=(_NBLK + 1,),
        in_specs=[
            pl.BlockSpec((_BLK, u_X.shape[1]),
                         lambda i: (jnp.minimum(i, last), 0)),
            pl.BlockSpec((_BLK, c_S.shape[1]),
                         lambda i: (jnp.minimum(i, last), 0)),
            pl.BlockSpec(W.shape, lambda i: (0, 0)),
            pl.BlockSpec((1, W.shape[0]), lambda i: (0, 0)),
            pl.BlockSpec(keys.shape, lambda i: (0, 0)),
            pl.BlockSpec(semantic_embeddings.shape, lambda i: (0, 0)),
            pl.BlockSpec(values.shape, lambda i: (0, 0)),
        ],
        out_specs=[
            pl.BlockSpec((_BLK, dv), lambda i: (jnp.maximum(i - 1, 0), 0)),
            pl.BlockSpec((_BLK, 8), lambda i: (jnp.maximum(i - 1, 0), 0)),
            pl.BlockSpec((1, 1), lambda i: (0, 0)),
        ],
        out_shape=[
            jax.ShapeDtypeStruct((_B, dv), jnp.float32),
            jax.ShapeDtypeStruct((_B, 8), jnp.int32),
            jax.ShapeDtypeStruct((1, 1), jnp.float32),
        ],
        scratch_shapes=[
            pltpu.VMEM((_KB_PAD, dk), jnp.float32),
            pltpu.VMEM((_KB_PAD, dv + dsem), jnp.float32),
            pltpu.VMEM((2, _BLK, _KB_PAD), jnp.float32),
            pltpu.VMEM((2, _BLK, 512), jnp.float32),
        ],
    )(u_X, c_S, W, b2, keys, semantic_embeddings, values)

    return (e_out, idx_out[:, :_TOPK], cost_out[0, 0])
